# Initial kernel scaffold; baseline (speedup 1.0000x reference)
import jax
import jax.numpy as jnp
from jax.experimental import pallas as pl


def kernel(packed, input_ids, eps):
    raise NotImplementedError("write your pallas kernel here")


# speedup vs baseline: 1.1473x; 1.1473x over previous
import functools

import jax
import jax.numpy as jnp
from jax import lax
from jax.experimental import pallas as pl
from jax.experimental.pallas import tpu as pltpu


def _round8(x):
    return ((x + 7) // 8) * 8


def _fused_kernel(
    ids_ref,
    pblk_ref,
    packed_hbm, eps_hbm,
    kl_ref,
    emb_ref,
    pk_buf, eps_buf,
    sems,
    *, T, tile_v, n_steps, V, D,
):
    c = pl.program_id(0)
    i = pl.program_id(1)
    slot = i % 2

    def issue(tile, dst_slot):
        base = tile * T

        def body(t, carry):
            row = ids_ref[base + t]
            pltpu.make_async_copy(
                packed_hbm.at[pl.ds(row, 1), :],
                pk_buf.at[dst_slot, pl.ds(t, 1), :],
                sems.at[dst_slot, 0]).start()
            pltpu.make_async_copy(
                eps_hbm.at[pl.ds(row, 1), :],
                eps_buf.at[dst_slot, pl.ds(t, 1), :],
                sems.at[dst_slot, 1]).start()
            return carry

        lax.fori_loop(0, T, body, 0, unroll=8)

    @pl.when(i == 0)
    def _():
        issue(c * n_steps, 0)

    @pl.when(i + 1 < n_steps)
    def _():
        issue(c * n_steps + i + 1, 1 - slot)

    blk = pblk_ref[...].astype(jnp.float32)
    mu = blk[:, :D]
    ls = blk[:, D:]
    kl = ls + 0.5 * (1.0 + mu * mu) * jnp.exp(-2.0 * ls) - 0.5
    start = (c * n_steps + i) * tile_v
    rows = start + lax.broadcasted_iota(jnp.int32, kl.shape, 0)
    kl = jnp.where(rows < V, kl, 0.0)
    part = jnp.sum(kl, axis=0, keepdims=True)[None]

    @pl.when(i == 0)
    def _():
        kl_ref[...] = jnp.zeros_like(kl_ref)

    kl_ref[...] = kl_ref[...] + part

    pltpu.make_async_copy(pk_buf.at[slot], pk_buf.at[slot], sems.at[slot, 0]).wait()
    pltpu.make_async_copy(eps_buf.at[slot], eps_buf.at[slot], sems.at[slot, 1]).wait()

    pk = pk_buf[slot].astype(jnp.float32)
    emb = pk[:, :D] + jnp.exp(pk[:, D:]) * eps_buf[slot].astype(jnp.float32)
    emb_ref[...] = emb.astype(emb_ref.dtype)


def kernel(packed, input_ids, eps):
    V, two_d = packed.shape
    D = two_d // 2
    B, S = input_ids.shape
    N = B * S

    n_steps = 32
    n_tiles = 2 * n_steps

    T = _round8(pl.cdiv(N, n_tiles))
    Np = n_tiles * T
    ids = input_ids.reshape(-1).astype(jnp.int32)
    if Np != N:
        ids = jnp.pad(ids, (0, Np - N))
    ids = jnp.clip(ids, 0, V - 1)

    tile_v = _round8(pl.cdiv(V, n_tiles))
    n_vblocks = pl.cdiv(V, tile_v)

    kl_part, emb = pl.pallas_call(
        functools.partial(_fused_kernel, T=T, tile_v=tile_v,
                          n_steps=n_steps, V=V, D=D),
        out_shape=[
            jax.ShapeDtypeStruct((2, 1, D), jnp.float32),
            jax.ShapeDtypeStruct((Np, D), packed.dtype),
        ],
        grid_spec=pltpu.PrefetchScalarGridSpec(
            num_scalar_prefetch=1,
            grid=(2, n_steps),
            in_specs=[
                pl.BlockSpec(
                    (tile_v, two_d),
                    lambda c, i, ids: (jnp.minimum(c * n_steps + i, n_vblocks - 1), 0)),
                pl.BlockSpec(memory_space=pl.ANY),
                pl.BlockSpec(memory_space=pl.ANY),
            ],
            out_specs=[
                pl.BlockSpec((1, 1, D), lambda c, i, ids: (c, 0, 0)),
                pl.BlockSpec((T, D), lambda c, i, ids: (c * n_steps + i, 0)),
            ],
            scratch_shapes=[
                pltpu.VMEM((2, T, two_d), packed.dtype),
                pltpu.VMEM((2, T, D), eps.dtype),
                pltpu.SemaphoreType.DMA((2, 2)),
            ],
        ),
        compiler_params=pltpu.CompilerParams(
            dimension_semantics=("parallel", "arbitrary"),
            vmem_limit_bytes=40 * 1024 * 1024,
            disable_bounds_checks=True,
        ),
    )(ids, packed, packed, eps)

    elbo = jnp.sum(kl_part)
    return emb[:N].reshape(B, S, D), elbo
